# Initial kernel scaffold; baseline (speedup 1.0000x reference)
import functools

import jax
import jax.numpy as jnp
from jax import lax
from jax.experimental import pallas as pl
from jax.experimental.pallas import tpu as pltpu
from jax.experimental.pallas import tpu_sc as plsc

N = 10000
E = 320000
D = 128

NC = 2
NS = 16
NW = NC * NS
L = 16

CH = 128
NCHUNKS = E // CH
CHUNKS_PER_W = -(-NCHUNKS // NW)
ROWS_PER_TILE = N // NS
WB = 125


def _proj_body(x_ref, w_ref, a1_ref, a2_ref, z_ref, s_ref, t_ref):
    zb = jnp.dot(x_ref[...], w_ref[...], preferred_element_type=jnp.float32)
    z_ref[...] = zb
    s_ref[...] = jnp.dot(zb, a1_ref[...], preferred_element_type=jnp.float32)
    t_ref[...] = jnp.dot(zb, a2_ref[...], preferred_element_type=jnp.float32)


_BN = 400

_proj = pl.pallas_call(
    _proj_body,
    grid=(N // _BN,),
    in_specs=[
        pl.BlockSpec((_BN, D), lambda i: (i, 0)),
        pl.BlockSpec((D, D), lambda i: (0, 0)),
        pl.BlockSpec((D, 1), lambda i: (0, 0)),
        pl.BlockSpec((D, 1), lambda i: (0, 0)),
    ],
    out_specs=[
        pl.BlockSpec((_BN, D), lambda i: (i, 0)),
        pl.BlockSpec((_BN, 1), lambda i: (i, 0)),
        pl.BlockSpec((_BN, 1), lambda i: (i, 0)),
    ],
    out_shape=[
        jax.ShapeDtypeStruct((N, D), jnp.float32),
        jax.ShapeDtypeStruct((N, 1), jnp.float32),
        jax.ShapeDtypeStruct((N, 1), jnp.float32),
    ],
)


def _sc_body(z_hbm, s_hbm, t_hbm, src_hbm, dst_hbm,
             h_out, d_out,
             s_v, t_v, sidx, didx, exb, rows,
             h_sh, d_sh, sem):
    cid = lax.axis_index("c")
    sid = lax.axis_index("s")
    wid = sid * NC + cid

    def _zero_rows(r, carry):
        for j in range(D // L):
            rows[r, pl.ds(j * L, L)] = jnp.zeros((L,), jnp.float32)
        return carry

    lax.fori_loop(0, CH, _zero_rows, 0)

    def _zero_sv(i, carry):
        s_v[pl.ds(i * L, L)] = jnp.zeros((L,), jnp.float32)
        return carry

    lax.fori_loop(0, N // L, _zero_sv, 0)

    for k in range(ROWS_PER_TILE // WB):
        pltpu.sync_copy(rows.at[pl.ds(0, WB)],
                        h_sh.at[pl.ds(sid * ROWS_PER_TILE + k * WB, WB)])

    @pl.when(sid == 0)
    def _():
        pltpu.sync_copy(s_v, d_sh)

    plsc.subcore_barrier()

    pltpu.sync_copy(s_hbm, s_v)
    pltpu.sync_copy(t_hbm, t_v)

    def _chunk(k, carry):
        chunk = wid + k * NW

        @pl.when(chunk < NCHUNKS)
        def _():
            base = chunk * CH
            pltpu.sync_copy(src_hbm.at[pl.ds(base, CH)], sidx)
            pltpu.sync_copy(dst_hbm.at[pl.ds(base, CH)], didx)
            cp = pltpu.async_copy(z_hbm.at[sidx], rows, sem)
            for g in range(CH // L):
                iv = sidx[pl.ds(g * L, L)]
                dv = didx[pl.ds(g * L, L)]
                sv = plsc.load_gather(s_v, [iv])
                tv = plsc.load_gather(t_v, [dv])
                exb[pl.ds(g * L, L)] = jnp.exp(sv + tv)
            cp.wait()

            def _scale(r, c2):
                ex_s = plsc.load_gather(exb, [jnp.full((L,), r, jnp.int32)])
                for j in range(D // L):
                    rows[r, pl.ds(j * L, L)] = rows[r, pl.ds(j * L, L)] * ex_s
                return c2

            lax.fori_loop(0, CH, _scale, 0)
            pltpu.sync_copy(rows, h_sh.at[didx], add=True)
            pltpu.sync_copy(exb, d_sh.at[didx], add=True)

        return carry

    lax.fori_loop(0, CHUNKS_PER_W, _chunk, 0)
    plsc.subcore_barrier()

    for k in range(ROWS_PER_TILE // WB):
        r0 = sid * ROWS_PER_TILE + k * WB
        pltpu.sync_copy(h_sh.at[pl.ds(r0, WB)], rows.at[pl.ds(0, WB)])
        pltpu.sync_copy(rows.at[pl.ds(0, WB)], h_out.at[cid, pl.ds(r0, WB)])

    @pl.when(sid == 0)
    def _():
        pltpu.sync_copy(d_sh, s_v)
        pltpu.sync_copy(s_v, d_out.at[cid])


_sc_scatter = functools.partial(
    pl.kernel,
    out_type=[
        jax.ShapeDtypeStruct((NC, N, D), jnp.float32),
        jax.ShapeDtypeStruct((NC, N), jnp.float32),
    ],
    mesh=plsc.VectorSubcoreMesh(core_axis_name="c", subcore_axis_name="s"),
    scratch_types=[
        pltpu.VMEM((N,), jnp.float32),
        pltpu.VMEM((N,), jnp.float32),
        pltpu.VMEM((CH,), jnp.int32),
        pltpu.VMEM((CH,), jnp.int32),
        pltpu.VMEM((CH,), jnp.float32),
        pltpu.VMEM((CH, D), jnp.float32),
        pltpu.VMEM_SHARED((N, D), jnp.float32),
        pltpu.VMEM_SHARED((N,), jnp.float32),
        pltpu.SemaphoreType.DMA,
    ],
)(_sc_body)


def _comb_body(hp_ref, dp_ref, o_ref):
    num = hp_ref[0] + hp_ref[1]
    den = dp_ref[0] + dp_ref[1]
    o_ref[...] = num / jnp.maximum(den, 1e-16)


_combine = pl.pallas_call(
    _comb_body,
    grid=(N // _BN,),
    in_specs=[
        pl.BlockSpec((NC, _BN, D), lambda i: (0, i, 0)),
        pl.BlockSpec((NC, _BN, 1), lambda i: (0, i, 0)),
    ],
    out_specs=pl.BlockSpec((_BN, D), lambda i: (i, 0)),
    out_shape=jax.ShapeDtypeStruct((N, D), jnp.float32),
)


def kernel(x, edge_index, W, a):
    src = edge_index[0]
    dst = edge_index[1]
    z, s1, t1 = _proj(x, W, a[:D], a[D:])
    hp, dp = _sc_scatter(z, s1.reshape(N), t1.reshape(N), src, dst)
    return _combine(hp, dp.reshape(NC, N, 1))


# speedup vs baseline: 17.7070x; 17.7070x over previous
import functools

import jax
import jax.numpy as jnp
from jax import lax
from jax.experimental import pallas as pl
from jax.experimental.pallas import tpu as pltpu
from jax.experimental.pallas import tpu_sc as plsc

N = 10000
E = 320000
D = 128

NC = 2
NS = 16
NW = NC * NS
L = 16

CH = 128
NCHUNKS = E // CH
CHUNKS_PER_W = -(-NCHUNKS // NW)
ROWS_PER_TILE = 624
_WCHUNKS = [(0, 128), (128, 128), (256, 128), (384, 128), (512, 112)]
_TAIL0 = NS * ROWS_PER_TILE
_TAILN = N - _TAIL0


def _proj_body(x_ref, w_ref, a1_ref, a2_ref, z_ref, s_ref, t_ref):
    zb = jnp.dot(x_ref[...], w_ref[...], preferred_element_type=jnp.float32)
    z_ref[...] = zb
    s_ref[...] = jnp.dot(zb, a1_ref[...], preferred_element_type=jnp.float32)
    t_ref[...] = jnp.dot(zb, a2_ref[...], preferred_element_type=jnp.float32)


_BN = 400

_proj = pl.pallas_call(
    _proj_body,
    grid=(N // _BN,),
    in_specs=[
        pl.BlockSpec((_BN, D), lambda i: (i, 0)),
        pl.BlockSpec((D, D), lambda i: (0, 0)),
        pl.BlockSpec((D, 1), lambda i: (0, 0)),
        pl.BlockSpec((D, 1), lambda i: (0, 0)),
    ],
    out_specs=[
        pl.BlockSpec((_BN, D), lambda i: (i, 0)),
        pl.BlockSpec((_BN, 1), lambda i: (i, 0)),
        pl.BlockSpec((_BN, 1), lambda i: (i, 0)),
    ],
    out_shape=[
        jax.ShapeDtypeStruct((N, D), jnp.float32),
        jax.ShapeDtypeStruct((N, 1), jnp.float32),
        jax.ShapeDtypeStruct((N, 1), jnp.float32),
    ],
)


def _sc_body(z_hbm, s_hbm, t_hbm, src_hbm, dst_hbm,
             h_out, d_out,
             s_v, t_v, sidx, didx, exb, rows,
             h_sh, d_sh, sem):
    cid = lax.axis_index("c")
    sid = lax.axis_index("s")
    wid = sid * NC + cid

    def _zero_rows(r, carry):
        for j in range(D // L):
            rows[r, pl.ds(j * L, L)] = jnp.zeros((L,), jnp.float32)
        return carry

    lax.fori_loop(0, CH, _zero_rows, 0)

    def _zero_sv(i, carry):
        s_v[pl.ds(i * L, L)] = jnp.zeros((L,), jnp.float32)
        return carry

    lax.fori_loop(0, N // L, _zero_sv, 0)

    for off, sz in _WCHUNKS:
        pltpu.sync_copy(rows.at[pl.ds(0, sz)],
                        h_sh.at[pl.ds(sid * ROWS_PER_TILE + off, sz)])

    @pl.when(sid == 0)
    def _():
        pltpu.sync_copy(rows.at[pl.ds(0, _TAILN)], h_sh.at[pl.ds(_TAIL0, _TAILN)])
        pltpu.sync_copy(s_v, d_sh)

    plsc.subcore_barrier()

    pltpu.sync_copy(s_hbm, s_v)
    pltpu.sync_copy(t_hbm, t_v)

    def _chunk(k, carry):
        chunk = wid + k * NW

        @pl.when(chunk < NCHUNKS)
        def _():
            base = chunk * CH
            pltpu.sync_copy(src_hbm.at[pl.ds(base, CH)], sidx)
            pltpu.sync_copy(dst_hbm.at[pl.ds(base, CH)], didx)
            cp = pltpu.async_copy(z_hbm.at[sidx], rows, sem)
            for g in range(CH // L):
                iv = sidx[pl.ds(g * L, L)]
                dv = didx[pl.ds(g * L, L)]
                sv = plsc.load_gather(s_v, [iv])
                tv = plsc.load_gather(t_v, [dv])
                exb[pl.ds(g * L, L)] = jnp.exp(sv + tv)
            cp.wait()

            def _scale(r, c2):
                ex_s = plsc.load_gather(exb, [jnp.full((L,), r, jnp.int32)])
                for j in range(D // L):
                    rows[r, pl.ds(j * L, L)] = rows[r, pl.ds(j * L, L)] * ex_s
                return c2

            lax.fori_loop(0, CH, _scale, 0)
            pltpu.sync_copy(rows, h_sh.at[didx], add=True)
            pltpu.sync_copy(exb, d_sh.at[didx], add=True)

        return carry

    lax.fori_loop(0, CHUNKS_PER_W, _chunk, 0)
    plsc.subcore_barrier()

    for off, sz in _WCHUNKS:
        r0 = sid * ROWS_PER_TILE + off
        pltpu.sync_copy(h_sh.at[pl.ds(r0, sz)], rows.at[pl.ds(0, sz)])
        pltpu.sync_copy(rows.at[pl.ds(0, sz)], h_out.at[cid, pl.ds(r0, sz)])

    @pl.when(sid == 0)
    def _():
        pltpu.sync_copy(h_sh.at[pl.ds(_TAIL0, _TAILN)], rows.at[pl.ds(0, _TAILN)])
        pltpu.sync_copy(rows.at[pl.ds(0, _TAILN)], h_out.at[cid, pl.ds(_TAIL0, _TAILN)])
        pltpu.sync_copy(d_sh, s_v)
        pltpu.sync_copy(s_v, d_out.at[cid])


_sc_scatter = functools.partial(
    pl.kernel,
    out_type=[
        jax.ShapeDtypeStruct((NC, N, D), jnp.float32),
        jax.ShapeDtypeStruct((NC, N), jnp.float32),
    ],
    mesh=plsc.VectorSubcoreMesh(core_axis_name="c", subcore_axis_name="s"),
    compiler_params=pltpu.CompilerParams(needs_layout_passes=False),
    scratch_types=[
        pltpu.VMEM((N,), jnp.float32),
        pltpu.VMEM((N,), jnp.float32),
        pltpu.VMEM((CH,), jnp.int32),
        pltpu.VMEM((CH,), jnp.int32),
        pltpu.VMEM((CH,), jnp.float32),
        pltpu.VMEM((CH, D), jnp.float32),
        pltpu.VMEM_SHARED((N, D), jnp.float32),
        pltpu.VMEM_SHARED((N,), jnp.float32),
        pltpu.SemaphoreType.DMA,
    ],
)(_sc_body)


def _comb_body(hp_ref, dp_ref, o_ref):
    num = hp_ref[0] + hp_ref[1]
    den = dp_ref[0] + dp_ref[1]
    o_ref[...] = num / jnp.maximum(den, 1e-16)


_combine = pl.pallas_call(
    _comb_body,
    grid=(N // _BN,),
    in_specs=[
        pl.BlockSpec((NC, _BN, D), lambda i: (0, i, 0)),
        pl.BlockSpec((NC, _BN, 1), lambda i: (0, i, 0)),
    ],
    out_specs=pl.BlockSpec((_BN, D), lambda i: (i, 0)),
    out_shape=jax.ShapeDtypeStruct((N, D), jnp.float32),
)


def kernel(x, edge_index, W, a):
    src = edge_index[0]
    dst = edge_index[1]
    z, s1, t1 = _proj(x, W, a[:D], a[D:])
    hp, dp = _sc_scatter(z, s1.reshape(N), t1.reshape(N), src, dst)
    return _combine(hp, dp.reshape(NC, N, 1))
